# baseline (device time: 40537 ns/iter reference)
import jax
import jax.numpy as jnp
from jax import lax
from jax.experimental import pallas as pl
from jax.experimental.pallas import tpu as pltpu

K = 32
BM = 128


def _bitonic_resort32_desc(c):
    iota = lax.broadcasted_iota(jnp.int32, c.shape, 1)
    for s in (16, 8, 4, 2, 1):
        left = jnp.concatenate([c[:, s:], c[:, :s]], axis=1)
        right = jnp.concatenate([c[:, -s:], c[:, :-s]], axis=1)
        c = jnp.where(
            (iota & s) == 0, jnp.maximum(c, left), jnp.minimum(c, right)
        )
    return c


def _lane_rev32(b):
    iota = lax.broadcasted_iota(jnp.int32, b.shape, 1)
    for s in (16, 8, 4, 2, 1):
        left = jnp.concatenate([b[:, s:], b[:, :s]], axis=1)
        right = jnp.concatenate([b[:, -s:], b[:, :-s]], axis=1)
        b = jnp.where((iota & s) == 0, left, right)
    return b


def _merge_sorted32_desc(a, b):
    return _bitonic_resort32_desc(jnp.maximum(a, _lane_rev32(b)))


def _topk_desc(xv, k):
    m = jnp.max(xv, axis=1, keepdims=True)
    outs = [m]
    for _ in range(k - 1):
        m = jnp.max(jnp.where(xv < m, xv, -jnp.inf), axis=1, keepdims=True)
        outs.append(m)
    return jnp.concatenate(outs, axis=1)


LANES = 128
PER_LANE = 8


def _local_topk_body(y_ref, x_ref, out_ref):
    del y_ref
    n_tiles = x_ref.shape[1] // LANES
    top = [x_ref[:, t * LANES : (t + 1) * LANES] for t in range(PER_LANE)]
    for j in range(PER_LANE):
        for i in range(PER_LANE - 1 - j):
            hi = jnp.maximum(top[i], top[i + 1])
            lo = jnp.minimum(top[i], top[i + 1])
            top[i], top[i + 1] = hi, lo
    for t in range(PER_LANE, n_tiles):
        v = x_ref[:, t * LANES : (t + 1) * LANES]
        for j in range(PER_LANE):
            hi = jnp.maximum(top[j], v)
            v = jnp.minimum(top[j], v)
            top[j] = hi
    half = LANES // 2
    rolled = [
        jnp.concatenate([t[:, half:], t[:, :half]], axis=1) for t in top
    ]
    c = [
        jnp.maximum(top[j], rolled[PER_LANE - 1 - j]) for j in range(PER_LANE)
    ]
    for s in (4, 2, 1):
        for i in range(PER_LANE):
            if (i & s) == 0:
                hi = jnp.maximum(c[i], c[i + s])
                lo = jnp.minimum(c[i], c[i + s])
                c[i], c[i + s] = hi, lo
    cand = jnp.concatenate([t[:, :half] for t in c], axis=1)
    out_ref[...] = _topk_desc(cand, K)


def _merge_body(loc_ref, out_ref, comm_ref, sx_send, sx_recv, sy_send, sy_recv):
    my_x = lax.axis_index("x")
    my_y = lax.axis_index("y")
    x_peer = (1 - my_x, my_y)
    y_peer = (my_x, 1 - my_y)
    half_m = loc_ref.shape[0]

    barrier_sem = pltpu.get_barrier_semaphore()
    for peer in (x_peer, y_peer):
        pl.semaphore_signal(
            barrier_sem, inc=1, device_id=peer,
            device_id_type=pl.DeviceIdType.MESH,
        )
    pl.semaphore_wait(barrier_sem, 2)

    rdma1 = pltpu.make_async_remote_copy(
        src_ref=loc_ref,
        dst_ref=comm_ref,
        send_sem=sx_send,
        recv_sem=sx_recv,
        device_id=x_peer,
        device_id_type=pl.DeviceIdType.MESH,
    )
    rdma1.start()
    rdma1.wait()

    row0 = my_y * half_m
    out_ref[pl.ds(row0, half_m), :] = _merge_sorted32_desc(
        loc_ref[...], comm_ref[...]
    )

    rdma2 = pltpu.make_async_remote_copy(
        src_ref=out_ref.at[pl.ds(row0, half_m), :],
        dst_ref=out_ref.at[pl.ds(row0, half_m), :],
        send_sem=sy_send,
        recv_sem=sy_recv,
        device_id=y_peer,
        device_id_type=pl.DeviceIdType.MESH,
    )
    rdma2.start()
    rdma2.wait()


def kernel(x):
    m, n_loc = x.shape
    half_m = m // 2
    n_blocks = half_m // BM

    my_y = jnp.full((1,), lax.axis_index("y"), jnp.int32)

    loc = pl.pallas_call(
        _local_topk_body,
        grid_spec=pltpu.PrefetchScalarGridSpec(
            num_scalar_prefetch=1,
            grid=(n_blocks,),
            in_specs=[
                pl.BlockSpec((BM, n_loc), lambda i, y: (y[0] * n_blocks + i, 0))
            ],
            out_specs=pl.BlockSpec((BM, K), lambda i, y: (i, 0)),
        ),
        out_shape=jax.ShapeDtypeStruct((half_m, K), jnp.float32),
    )(my_y, x)

    return pl.pallas_call(
        _merge_body,
        out_shape=jax.ShapeDtypeStruct((m, K), jnp.float32),
        in_specs=[pl.BlockSpec(memory_space=pltpu.VMEM)],
        out_specs=pl.BlockSpec(memory_space=pltpu.VMEM),
        scratch_shapes=[
            pltpu.VMEM((half_m, K), jnp.float32),
            pltpu.SemaphoreType.DMA,
            pltpu.SemaphoreType.DMA,
            pltpu.SemaphoreType.DMA,
            pltpu.SemaphoreType.DMA,
        ],
        compiler_params=pltpu.CompilerParams(collective_id=0),
    )(loc)


# device time: 32831 ns/iter; 1.2347x vs baseline; 1.2347x over previous
import jax
import jax.numpy as jnp
from jax import lax
from jax.experimental import pallas as pl
from jax.experimental.pallas import tpu as pltpu

K = 32
BM = 128
LANES = 128
PER_LANE = 8


def _bitonic_resort32_desc(c):
    iota = lax.broadcasted_iota(jnp.int32, c.shape, 1)
    for s in (16, 8, 4, 2, 1):
        left = jnp.concatenate([c[:, s:], c[:, :s]], axis=1)
        right = jnp.concatenate([c[:, -s:], c[:, :-s]], axis=1)
        c = jnp.where(
            (iota & s) == 0, jnp.maximum(c, left), jnp.minimum(c, right)
        )
    return c


def _lane_rev32(b):
    iota = lax.broadcasted_iota(jnp.int32, b.shape, 1)
    for s in (16, 8, 4, 2, 1):
        left = jnp.concatenate([b[:, s:], b[:, :s]], axis=1)
        right = jnp.concatenate([b[:, -s:], b[:, :-s]], axis=1)
        b = jnp.where((iota & s) == 0, left, right)
    return b


def _merge_sorted32_desc(a, b):
    return _bitonic_resort32_desc(jnp.maximum(a, _lane_rev32(b)))


def _topk_desc(xv, k):
    m = jnp.max(xv, axis=1, keepdims=True)
    outs = [m]
    for _ in range(k - 1):
        m = jnp.max(jnp.where(xv < m, xv, -jnp.inf), axis=1, keepdims=True)
        outs.append(m)
    return jnp.concatenate(outs, axis=1)


def _local_topk(x_blk):
    n_tiles = x_blk.shape[1] // LANES
    top = [x_blk[:, t * LANES : (t + 1) * LANES] for t in range(PER_LANE)]
    for j in range(PER_LANE):
        for i in range(PER_LANE - 1 - j):
            hi = jnp.maximum(top[i], top[i + 1])
            lo = jnp.minimum(top[i], top[i + 1])
            top[i], top[i + 1] = hi, lo
    for t in range(PER_LANE, n_tiles):
        v = x_blk[:, t * LANES : (t + 1) * LANES]
        for j in range(PER_LANE):
            hi = jnp.maximum(top[j], v)
            v = jnp.minimum(top[j], v)
            top[j] = hi
    cand = jnp.concatenate(top, axis=1)
    return _topk_desc(cand, K)


def kernel(x):
    m, n_loc = x.shape
    half_m = m // 2
    nb = half_m // BM

    my_y_arr = jnp.full((1,), lax.axis_index("y"), jnp.int32)

    def body(y_ref, x_ref, out_ref, loc_buf, comm_buf, xs_sems, xr_sems,
             ys_sems, yr_sems):
        i = pl.program_id(0)
        my_x = lax.axis_index("x")
        my_y = y_ref[0]
        x_peer = (1 - my_x, my_y)
        y_peer = (my_x, 1 - my_y)

        def x_rdma(b):
            return pltpu.make_async_remote_copy(
                src_ref=loc_buf.at[b],
                dst_ref=comm_buf.at[b],
                send_sem=xs_sems.at[b],
                recv_sem=xr_sems.at[b],
                device_id=x_peer,
                device_id_type=pl.DeviceIdType.MESH,
            )

        def y_rdma(b):
            row0 = my_y * half_m + b * BM
            return pltpu.make_async_remote_copy(
                src_ref=out_ref.at[pl.ds(row0, BM), :],
                dst_ref=out_ref.at[pl.ds(row0, BM), :],
                send_sem=ys_sems.at[b],
                recv_sem=yr_sems.at[b],
                device_id=y_peer,
                device_id_type=pl.DeviceIdType.MESH,
            )

        @pl.when(i == 0)
        def _():
            barrier_sem = pltpu.get_barrier_semaphore()
            for peer in (x_peer, y_peer):
                pl.semaphore_signal(
                    barrier_sem, inc=1, device_id=peer,
                    device_id_type=pl.DeviceIdType.MESH,
                )
            pl.semaphore_wait(barrier_sem, 2)

        @pl.when(i < nb)
        def _():
            loc_buf[i] = _local_topk(x_ref[...])
            x_rdma(i).start()

        @pl.when(i > 0)
        def _():
            b = i - 1
            x_rdma(b).wait_recv()
            merged = _merge_sorted32_desc(loc_buf[b], comm_buf[b])
            row0 = my_y * half_m + b * BM
            out_ref[pl.ds(row0, BM), :] = merged
            y_rdma(b).start()

        @pl.when(i == nb)
        def _():
            for b in range(nb):
                x_rdma(b).wait_send()
                yd = y_rdma(b)
                yd.wait_send()
                yd.wait_recv()

    return pl.pallas_call(
        body,
        grid_spec=pltpu.PrefetchScalarGridSpec(
            num_scalar_prefetch=1,
            grid=(nb + 1,),
            in_specs=[
                pl.BlockSpec(
                    (BM, n_loc),
                    lambda i, y: (y[0] * nb + jnp.minimum(i, nb - 1), 0),
                )
            ],
            out_specs=pl.BlockSpec((m, K), lambda i, y: (0, 0)),
            scratch_shapes=[
                pltpu.VMEM((nb, BM, K), jnp.float32),
                pltpu.VMEM((nb, BM, K), jnp.float32),
                pltpu.SemaphoreType.DMA((nb,)),
                pltpu.SemaphoreType.DMA((nb,)),
                pltpu.SemaphoreType.DMA((nb,)),
                pltpu.SemaphoreType.DMA((nb,)),
            ],
        ),
        out_shape=jax.ShapeDtypeStruct((m, K), jnp.float32),
        compiler_params=pltpu.CompilerParams(collective_id=0),
    )(my_y_arr, x)


# device time: 28095 ns/iter; 1.4429x vs baseline; 1.1686x over previous
import jax
import jax.numpy as jnp
from jax import lax
from jax.experimental import pallas as pl
from jax.experimental.pallas import tpu as pltpu

K = 32
BM = 128
LANES = 128
PER_LANE = 4


def _bitonic_resort32_desc(c):
    iota = lax.broadcasted_iota(jnp.int32, c.shape, 1)
    for s in (16, 8, 4, 2, 1):
        left = jnp.concatenate([c[:, s:], c[:, :s]], axis=1)
        right = jnp.concatenate([c[:, -s:], c[:, :-s]], axis=1)
        c = jnp.where(
            (iota & s) == 0, jnp.maximum(c, left), jnp.minimum(c, right)
        )
    return c


def _lane_rev32(b):
    iota = lax.broadcasted_iota(jnp.int32, b.shape, 1)
    for s in (16, 8, 4, 2, 1):
        left = jnp.concatenate([b[:, s:], b[:, :s]], axis=1)
        right = jnp.concatenate([b[:, -s:], b[:, :-s]], axis=1)
        b = jnp.where((iota & s) == 0, left, right)
    return b


def _merge_sorted32_desc(a, b):
    return _bitonic_resort32_desc(jnp.maximum(a, _lane_rev32(b)))


def _topk_desc(xv, k):
    m = jnp.max(xv, axis=1, keepdims=True)
    outs = [m]
    for _ in range(k - 1):
        m = jnp.max(jnp.where(xv < m, xv, -jnp.inf), axis=1, keepdims=True)
        outs.append(m)
    return jnp.concatenate(outs, axis=1)


def _local_topk(x_blk):
    n_tiles = x_blk.shape[1] // LANES
    top = [x_blk[:, t * LANES : (t + 1) * LANES] for t in range(PER_LANE)]
    for j in range(PER_LANE):
        for i in range(PER_LANE - 1 - j):
            hi = jnp.maximum(top[i], top[i + 1])
            lo = jnp.minimum(top[i], top[i + 1])
            top[i], top[i + 1] = hi, lo
    for t in range(PER_LANE, n_tiles):
        v = x_blk[:, t * LANES : (t + 1) * LANES]
        for j in range(PER_LANE):
            hi = jnp.maximum(top[j], v)
            v = jnp.minimum(top[j], v)
            top[j] = hi
    cand = jnp.concatenate(top, axis=1)
    return _topk_desc(cand, K)


def kernel(x):
    m, n_loc = x.shape
    half_m = m // 2
    nb = half_m // BM

    my_y_arr = jnp.full((1,), lax.axis_index("y"), jnp.int32)

    def body(y_ref, x_ref, out_ref, loc_buf, comm_buf, xs_sems, xr_sems,
             ys_sems, yr_sems):
        i = pl.program_id(0)
        my_x = lax.axis_index("x")
        my_y = y_ref[0]
        x_peer = (1 - my_x, my_y)
        y_peer = (my_x, 1 - my_y)

        def x_rdma(b):
            return pltpu.make_async_remote_copy(
                src_ref=loc_buf.at[b],
                dst_ref=comm_buf.at[b],
                send_sem=xs_sems.at[b],
                recv_sem=xr_sems.at[b],
                device_id=x_peer,
                device_id_type=pl.DeviceIdType.MESH,
            )

        def y_rdma(b):
            row0 = my_y * half_m + b * BM
            return pltpu.make_async_remote_copy(
                src_ref=out_ref.at[pl.ds(row0, BM), :],
                dst_ref=out_ref.at[pl.ds(row0, BM), :],
                send_sem=ys_sems.at[b],
                recv_sem=yr_sems.at[b],
                device_id=y_peer,
                device_id_type=pl.DeviceIdType.MESH,
            )

        @pl.when(i == 0)
        def _():
            barrier_sem = pltpu.get_barrier_semaphore()
            for peer in (x_peer, y_peer):
                pl.semaphore_signal(
                    barrier_sem, inc=1, device_id=peer,
                    device_id_type=pl.DeviceIdType.MESH,
                )
            pl.semaphore_wait(barrier_sem, 2)

        @pl.when(i < nb)
        def _():
            loc_buf[i] = _local_topk(x_ref[...])
            x_rdma(i).start()

        @pl.when(i > 0)
        def _():
            b = i - 1
            x_rdma(b).wait_recv()
            merged = _merge_sorted32_desc(loc_buf[b], comm_buf[b])
            row0 = my_y * half_m + b * BM
            out_ref[pl.ds(row0, BM), :] = merged
            y_rdma(b).start()

        @pl.when(i == nb)
        def _():
            for b in range(nb):
                x_rdma(b).wait_send()
                yd = y_rdma(b)
                yd.wait_send()
                yd.wait_recv()

    return pl.pallas_call(
        body,
        grid_spec=pltpu.PrefetchScalarGridSpec(
            num_scalar_prefetch=1,
            grid=(nb + 1,),
            in_specs=[
                pl.BlockSpec(
                    (BM, n_loc),
                    lambda i, y: (y[0] * nb + jnp.minimum(i, nb - 1), 0),
                )
            ],
            out_specs=pl.BlockSpec((m, K), lambda i, y: (0, 0)),
            scratch_shapes=[
                pltpu.VMEM((nb, BM, K), jnp.float32),
                pltpu.VMEM((nb, BM, K), jnp.float32),
                pltpu.SemaphoreType.DMA((nb,)),
                pltpu.SemaphoreType.DMA((nb,)),
                pltpu.SemaphoreType.DMA((nb,)),
                pltpu.SemaphoreType.DMA((nb,)),
            ],
        ),
        out_shape=jax.ShapeDtypeStruct((m, K), jnp.float32),
        compiler_params=pltpu.CompilerParams(collective_id=0),
    )(my_y_arr, x)


# device time: 27998 ns/iter; 1.4479x vs baseline; 1.0035x over previous
import jax
import jax.numpy as jnp
from jax import lax
from jax.experimental import pallas as pl
from jax.experimental.pallas import tpu as pltpu

K = 32
BM = 128
LANES = 128
PER_LANE = 4


def _bitonic_resort32_desc(c):
    iota = lax.broadcasted_iota(jnp.int32, c.shape, 1)
    for s in (16, 8, 4, 2, 1):
        left = jnp.concatenate([c[:, s:], c[:, :s]], axis=1)
        right = jnp.concatenate([c[:, -s:], c[:, :-s]], axis=1)
        c = jnp.where(
            (iota & s) == 0, jnp.maximum(c, left), jnp.minimum(c, right)
        )
    return c


def _lane_rev32(b):
    iota = lax.broadcasted_iota(jnp.int32, b.shape, 1)
    for s in (16, 8, 4, 2, 1):
        left = jnp.concatenate([b[:, s:], b[:, :s]], axis=1)
        right = jnp.concatenate([b[:, -s:], b[:, :-s]], axis=1)
        b = jnp.where((iota & s) == 0, left, right)
    return b


def _merge_sorted32_desc(a, b):
    return _bitonic_resort32_desc(jnp.maximum(a, _lane_rev32(b)))


def _topk_desc(xv, k):
    m = jnp.max(xv, axis=1, keepdims=True)
    outs = [m]
    for _ in range(k - 1):
        m = jnp.max(jnp.where(xv < m, xv, -jnp.inf), axis=1, keepdims=True)
        outs.append(m)
    return jnp.concatenate(outs, axis=1)


N_CHAINS = 4


def _chain_topk(tiles):
    top = list(tiles[:PER_LANE])
    for j in range(PER_LANE):
        for i in range(PER_LANE - 1 - j):
            hi = jnp.maximum(top[i], top[i + 1])
            lo = jnp.minimum(top[i], top[i + 1])
            top[i], top[i + 1] = hi, lo
    for v in tiles[PER_LANE:]:
        for j in range(PER_LANE):
            hi = jnp.maximum(top[j], v)
            v = jnp.minimum(top[j], v)
            top[j] = hi
    return top


def _merge_chains(a, b):
    c = [jnp.maximum(a[i], b[PER_LANE - 1 - i]) for i in range(PER_LANE)]
    s = PER_LANE // 2
    while s >= 1:
        for i in range(PER_LANE):
            if (i & s) == 0:
                hi = jnp.maximum(c[i], c[i + s])
                lo = jnp.minimum(c[i], c[i + s])
                c[i], c[i + s] = hi, lo
        s //= 2
    return c


def _local_topk(x_blk):
    n_tiles = x_blk.shape[1] // LANES
    tiles = [x_blk[:, t * LANES : (t + 1) * LANES] for t in range(n_tiles)]
    per_chain = n_tiles // N_CHAINS
    chains = [
        _chain_topk(tiles[c * per_chain : (c + 1) * per_chain])
        for c in range(N_CHAINS)
    ]
    while len(chains) > 1:
        chains = [
            _merge_chains(chains[i], chains[i + 1])
            for i in range(0, len(chains), 2)
        ]
    cand = jnp.concatenate(chains[0], axis=1)
    return _topk_desc(cand, K)


def kernel(x):
    m, n_loc = x.shape
    half_m = m // 2
    nb = half_m // BM

    my_y_arr = jnp.full((1,), lax.axis_index("y"), jnp.int32)

    def body(y_ref, x_ref, out_ref, loc_buf, comm_buf, xs_sems, xr_sems,
             ys_sems, yr_sems):
        i = pl.program_id(0)
        my_x = lax.axis_index("x")
        my_y = y_ref[0]
        x_peer = (1 - my_x, my_y)
        y_peer = (my_x, 1 - my_y)

        def x_rdma(b):
            return pltpu.make_async_remote_copy(
                src_ref=loc_buf.at[b],
                dst_ref=comm_buf.at[b],
                send_sem=xs_sems.at[b],
                recv_sem=xr_sems.at[b],
                device_id=x_peer,
                device_id_type=pl.DeviceIdType.MESH,
            )

        def y_rdma(b):
            row0 = my_y * half_m + b * BM
            return pltpu.make_async_remote_copy(
                src_ref=out_ref.at[pl.ds(row0, BM), :],
                dst_ref=out_ref.at[pl.ds(row0, BM), :],
                send_sem=ys_sems.at[b],
                recv_sem=yr_sems.at[b],
                device_id=y_peer,
                device_id_type=pl.DeviceIdType.MESH,
            )

        @pl.when(i == 0)
        def _():
            barrier_sem = pltpu.get_barrier_semaphore()
            for peer in (x_peer, y_peer):
                pl.semaphore_signal(
                    barrier_sem, inc=1, device_id=peer,
                    device_id_type=pl.DeviceIdType.MESH,
                )
            pl.semaphore_wait(barrier_sem, 2)

        @pl.when(i < nb)
        def _():
            loc_buf[i] = _local_topk(x_ref[...])
            x_rdma(i).start()

        @pl.when(i > 0)
        def _():
            b = i - 1
            x_rdma(b).wait_recv()
            merged = _merge_sorted32_desc(loc_buf[b], comm_buf[b])
            row0 = my_y * half_m + b * BM
            out_ref[pl.ds(row0, BM), :] = merged
            y_rdma(b).start()

        @pl.when(i == nb)
        def _():
            for b in range(nb):
                x_rdma(b).wait_send()
                yd = y_rdma(b)
                yd.wait_send()
                yd.wait_recv()

    return pl.pallas_call(
        body,
        grid_spec=pltpu.PrefetchScalarGridSpec(
            num_scalar_prefetch=1,
            grid=(nb + 1,),
            in_specs=[
                pl.BlockSpec(
                    (BM, n_loc),
                    lambda i, y: (y[0] * nb + jnp.minimum(i, nb - 1), 0),
                )
            ],
            out_specs=pl.BlockSpec((m, K), lambda i, y: (0, 0)),
            scratch_shapes=[
                pltpu.VMEM((nb, BM, K), jnp.float32),
                pltpu.VMEM((nb, BM, K), jnp.float32),
                pltpu.SemaphoreType.DMA((nb,)),
                pltpu.SemaphoreType.DMA((nb,)),
                pltpu.SemaphoreType.DMA((nb,)),
                pltpu.SemaphoreType.DMA((nb,)),
            ],
        ),
        out_shape=jax.ShapeDtypeStruct((m, K), jnp.float32),
        compiler_params=pltpu.CompilerParams(collective_id=0),
    )(my_y_arr, x)
